# baseline (speedup 1.0000x reference)
import functools

import jax
import jax.numpy as jnp
from jax import lax
from jax.experimental import pallas as pl
from jax.experimental.pallas import tpu as pltpu
from jax.experimental.pallas import tpu_sc as plsc

NC = 2
NS = 16
NW = NC * NS


@functools.lru_cache(maxsize=None)
def _build_sc_kernel(B, L, D):
    PER_W = B // NW
    G = 8
    ROWS = G * L
    NCHUNK = PER_W // G
    NH = D // 32

    mesh = plsc.VectorSubcoreMesh(core_axis_name="c", subcore_axis_name="s")

    @functools.partial(
        pl.kernel,
        out_type=jax.ShapeDtypeStruct((B * L,), jnp.float32),
        mesh=mesh,
        compiler_params=pltpu.CompilerParams(
            needs_layout_passes=False, use_tc_tiling_on_sc=False),
        scratch_types=[
            pltpu.VMEM((PER_W, L), jnp.int32),
            pltpu.VMEM((PER_W,), jnp.int32),
            pltpu.VMEM((PER_W, D), jnp.bfloat16),
            pltpu.VMEM((ROWS, D), jnp.bfloat16),
            pltpu.VMEM((ROWS, D), jnp.bfloat16),
            pltpu.VMEM((PER_W * L + 16,), jnp.float32),
            pltpu.SemaphoreType.DMA,
            pltpu.SemaphoreType.DMA,
            pltpu.SemaphoreType.DMA,
        ],
    )
    def k(cen_hbm, ctx_hbm, embed_hbm, out_hbm,
          ctx_v, cen_v, u_v, vb0, vb1, out_v, sem0, sem1, semu):
        vbufs = (vb0, vb1)
        sems = (sem0, sem1)
        wid = lax.axis_index("s") * NC + lax.axis_index("c")
        base = wid * PER_W

        pltpu.sync_copy(ctx_hbm.at[pl.ds(base, PER_W)], ctx_v)
        pltpu.sync_copy(cen_hbm.at[pl.ds(base, PER_W)], cen_v)

        def fire(g, vb, sem):
            for c in range(G):
                pltpu.async_copy(embed_hbm.at[ctx_v.at[g * G + c]],
                                 vb.at[pl.ds(c * L, L)], sem)

        def drain(g, vb, sem):
            for c in range(G):
                pltpu.make_async_copy(embed_hbm.at[ctx_v.at[g * G + c]],
                                      vb.at[pl.ds(c * L, L)], sem).wait()

        NBUF = 2
        for b in range(NBUF):
            fire(b, vbufs[b], sems[b])
        NT = PER_W // 128
        for t in range(NT):
            pltpu.async_copy(embed_hbm.at[cen_v.at[pl.ds(t * 128, 128)]],
                             u_v.at[pl.ds(t * 128, 128)], semu)
        for t in range(NT):
            pltpu.make_async_copy(embed_hbm.at[cen_v.at[pl.ds(t * 128, 128)]],
                                  u_v.at[pl.ds(t * 128, 128)], semu).wait()

        lanes = lax.iota(jnp.int32, 16)

        def compute(g, vb):
            def center_body(c, carry):
                cg = g * G + c
                uf = []
                for kk in range(NH):
                    ua, ub = plsc.unpack(u_v[cg, pl.ds(kk * 32, 32)],
                                         format=plsc.PackFormat.INTERLEAVED)
                    uf += [ua, ub]
                for j0 in range(0, L, 16):
                    nv = min(L - j0, 16)
                    ovec = jnp.zeros((16,), jnp.float32)
                    for j in range(nv):
                        r = c * L + j0 + j
                        p = None
                        for kk in range(NH):
                            va, vbb = plsc.unpack(vb[r, pl.ds(kk * 32, 32)],
                                                  format=plsc.PackFormat.INTERLEAVED)
                            t = va * uf[2 * kk] + vbb * uf[2 * kk + 1]
                            p = t if p is None else p + t
                        ovec = jnp.where(lanes == j, jnp.sum(p), ovec)
                    out_v[pl.ds(cg * L + j0, 16)] = ovec
                return carry

            lax.fori_loop(0, G, center_body, 0)

        def body(pi, carry):
            g0 = pi * NBUF
            for b in range(NBUF):
                drain(g0 + b, vbufs[b], sems[b])
                compute(g0 + b, vbufs[b])
                fire(g0 + NBUF + b, vbufs[b], sems[b])
            return carry

        lax.fori_loop(0, NCHUNK // NBUF - 1, body, 0)
        gl = NCHUNK - NBUF
        for b in range(NBUF):
            drain(gl + b, vbufs[b], sems[b])
            compute(gl + b, vbufs[b])

        pltpu.sync_copy(out_v.at[pl.ds(0, PER_W * L)],
                        out_hbm.at[pl.ds(base * L, PER_W * L)])

    return k


def kernel(center, contexts, embed):
    B, L = contexts.shape
    _, D = embed.shape
    out = _build_sc_kernel(B, L, D)(
        center.reshape(B), contexts, embed.astype(jnp.bfloat16))
    return out.reshape(B, L, 1)


# speedup vs baseline: 1.2544x; 1.2544x over previous
import functools

import jax
import jax.numpy as jnp
from jax import lax
from jax.experimental import pallas as pl
from jax.experimental.pallas import tpu as pltpu
from jax.experimental.pallas import tpu_sc as plsc

NC = 2
NS = 16
NW = NC * NS


@functools.lru_cache(maxsize=None)
def _build_sc_kernel(B, L, D):
    PER_W = B // NW
    G = 4
    ROWS = G * L
    NCHUNK = PER_W // G
    NK = D // 16

    mesh = plsc.VectorSubcoreMesh(core_axis_name="c", subcore_axis_name="s")

    @functools.partial(
        pl.kernel,
        out_type=jax.ShapeDtypeStruct((B * L,), jnp.float32),
        mesh=mesh,
        compiler_params=pltpu.CompilerParams(
            needs_layout_passes=False, use_tc_tiling_on_sc=False),
        scratch_types=[
            pltpu.VMEM((PER_W * L,), jnp.int32),
            pltpu.VMEM((PER_W,), jnp.int32),
            pltpu.VMEM((PER_W, D), jnp.float32),
            pltpu.VMEM((ROWS, D), jnp.float32),
            pltpu.VMEM((ROWS, D), jnp.float32),
            pltpu.VMEM((PER_W * L + 16,), jnp.float32),
            pltpu.SemaphoreType.DMA,
            pltpu.SemaphoreType.DMA,
            pltpu.SemaphoreType.DMA,
        ],
    )
    def k(cen_hbm, ctx_hbm, embed_hbm, out_hbm,
          ctx_v, cen_v, u_v, vb0, vb1, out_v, sem0, sem1, semu):
        vbufs = (vb0, vb1)
        sems = (sem0, sem1)
        wid = lax.axis_index("s") * NC + lax.axis_index("c")
        base = wid * PER_W

        pltpu.sync_copy(ctx_hbm.at[pl.ds(base * L, PER_W * L)], ctx_v)
        pltpu.sync_copy(cen_hbm.at[pl.ds(base, PER_W)], cen_v)

        def fire(g, vb, sem):
            pltpu.async_copy(embed_hbm.at[ctx_v.at[pl.ds(g * ROWS, ROWS)]], vb, sem)

        def drain(g, vb, sem):
            pltpu.make_async_copy(
                embed_hbm.at[ctx_v.at[pl.ds(g * ROWS, ROWS)]], vb, sem).wait()

        NBUF = 2
        for b in range(NBUF):
            fire(b, vbufs[b], sems[b])
        NT = PER_W // 128
        for t in range(NT):
            pltpu.async_copy(embed_hbm.at[cen_v.at[pl.ds(t * 128, 128)]],
                             u_v.at[pl.ds(t * 128, 128)], semu)
        for t in range(NT):
            pltpu.make_async_copy(embed_hbm.at[cen_v.at[pl.ds(t * 128, 128)]],
                                  u_v.at[pl.ds(t * 128, 128)], semu).wait()

        lanes = lax.iota(jnp.int32, 16)

        def compute(g, vb):
            def center_body(c, carry):
                cg = g * G + c
                u = [u_v[cg, pl.ds(kk * 16, 16)] for kk in range(NK)]
                for j0 in range(0, L, 16):
                    nv = min(L - j0, 16)
                    ovec = jnp.zeros((16,), jnp.float32)
                    for j in range(nv):
                        r = c * L + j0 + j
                        p = vb[r, pl.ds(0, 16)] * u[0]
                        for kk in range(1, NK):
                            p = p + vb[r, pl.ds(kk * 16, 16)] * u[kk]
                        ovec = jnp.where(lanes == j, jnp.sum(p), ovec)
                    out_v[pl.ds(cg * L + j0, 16)] = ovec
                return carry

            lax.fori_loop(0, G, center_body, 0)

        def body(pi, carry):
            g0 = pi * NBUF
            for b in range(NBUF):
                drain(g0 + b, vbufs[b], sems[b])
                compute(g0 + b, vbufs[b])
                fire(g0 + NBUF + b, vbufs[b], sems[b])
            return carry

        lax.fori_loop(0, NCHUNK // NBUF - 1, body, 0)
        gl = NCHUNK - NBUF
        for b in range(NBUF):
            drain(gl + b, vbufs[b], sems[b])
            compute(gl + b, vbufs[b])

        pltpu.sync_copy(out_v.at[pl.ds(0, PER_W * L)],
                        out_hbm.at[pl.ds(base * L, PER_W * L)])

    return k


def kernel(center, contexts, embed):
    B, L = contexts.shape
    _, D = embed.shape
    out = _build_sc_kernel(B, L, D)(
        center.reshape(B), contexts.reshape(B * L), embed)
    return out.reshape(B, L, 1)
